# baseline (speedup 1.0000x reference)
import functools

import jax
import jax.numpy as jnp
from jax import lax
from jax.experimental import pallas as pl
from jax.experimental.pallas import tpu as pltpu
from jax.experimental.pallas import tpu_sc as plsc

B = 16384
D = 64
N = 1_000_000

_info = plsc.get_sparse_core_info()
_NC = _info.num_cores
_NS = _info.num_subcores
_NW = _NC * _NS
_BPW = B // _NW

_RW = 16384
_H = _RW // 2
_NBLK = (N + _RW - 1) // _RW
_QN = _NBLK * _H


def _reformat_body(mlp_t, mf_t, w0h, out):
    dn = (((0,), (0,)), ((), ()))
    r = jax.lax.broadcasted_iota(jnp.int32, (D, D), 0)
    c = jax.lax.broadcasted_iota(jnp.int32, (D, D), 1)
    eye = (r == c).astype(jnp.float32)

    def b16(x):
        return jax.lax.bitcast_convert_type(
            x.astype(jnp.bfloat16), jnp.int16).astype(jnp.int32)

    CH = _H // 2
    for k in range(2):
        for rhs, col in ((w0h[...], 0), (eye, D)):
            src = mlp_t if col == 0 else mf_t
            lo = jax.lax.dot_general(
                src[:, pl.ds(k * CH, CH)], rhs, dn,
                preferred_element_type=jnp.float32)
            hi = jax.lax.dot_general(
                src[:, pl.ds(k * CH + _H, CH)], rhs, dn,
                preferred_element_type=jnp.float32)
            out[pl.ds(k * CH, CH), pl.ds(col, D)] = (
                (b16(hi) << 16) | (b16(lo) & 0xFFFF))


def _reformat(emb_mlp_t, emb_mf_t, w0h):
    in_spec = pl.BlockSpec((D, _RW), lambda i: (0, i))
    return pl.pallas_call(
        _reformat_body,
        grid=(pl.cdiv(N, _RW),),
        in_specs=[in_spec, in_spec,
                  pl.BlockSpec((D, D), lambda i: (0, 0))],
        out_specs=pl.BlockSpec((_H, 2 * D), lambda i: (i, 0)),
        out_shape=jax.ShapeDtypeStruct((_QN, 2 * D), jnp.int32),
        compiler_params=pltpu.CompilerParams(
            vmem_limit_bytes=100 * 1024 * 1024),
    )(emb_mlp_t, emb_mf_t, w0h)


_HC = _BPW // 2


def _sc_gather2(uidx_hbm, iidx_hbm, t_user, t_item, o_user, o_item,
                uidx_v, iidx_v, rows_a, rows_b, sem_a, sem_b):
    wid = lax.axis_index("s") * _NC + lax.axis_index("c")
    base = wid * _BPW
    pltpu.sync_copy(uidx_hbm.at[pl.ds(base, _BPW)], uidx_v)
    pltpu.sync_copy(iidx_hbm.at[pl.ds(base, _BPW)], iidx_v)
    work = ((t_user, uidx_v, o_user, 0), (t_item, iidx_v, o_item, 0),
            (t_user, uidx_v, o_user, 1), (t_item, iidx_v, o_item, 1))
    bufs = (rows_a, rows_b)
    sems = (sem_a, sem_b)
    pend = [None, None]
    for slot, (table, idx_v, out, half) in enumerate(work):
        s = slot % 2
        if pend[s] is not None:
            pend[s].wait()
        cp = pltpu.async_copy(
            table.at[idx_v.at[pl.ds(half * _HC, _HC)]], bufs[s], sems[s])
        cp.wait()
        pend[s] = pltpu.async_copy(
            bufs[s], out.at[pl.ds(base + half * _HC, _HC)], sems[s])
    for p in pend:
        p.wait()


_gather2 = functools.partial(
    pl.kernel,
    mesh=plsc.VectorSubcoreMesh(core_axis_name="c", subcore_axis_name="s"),
    out_type=[jax.ShapeDtypeStruct((B, 2 * D), jnp.int32)] * 2,
    scratch_types=[
        pltpu.VMEM((_BPW,), jnp.int32),
        pltpu.VMEM((_BPW,), jnp.int32),
        pltpu.VMEM((_BPW // 2, 2 * D), jnp.int32),
        pltpu.VMEM((_BPW // 2, 2 * D), jnp.int32),
        pltpu.SemaphoreType.DMA,
        pltpu.SemaphoreType.DMA,
    ],
)(_sc_gather2)


_BS = 2048


def _unpack(packed, sel):
    lo = jax.lax.bitcast_convert_type(
        jax.lax.shift_left(packed, 16), jnp.float32)
    hi = jax.lax.bitcast_convert_type(packed & jnp.int32(-65536), jnp.float32)
    return jnp.where(sel != 0, hi, lo)


def _mlp_body(u_pack, u_sel, i_pack, i_sel, b0, w1, b1, w2, b2,
              wtop, wbot, bout, out):
    u_cat = _unpack(u_pack[...], u_sel[...])
    i_cat = _unpack(i_pack[...], i_sel[...])
    h = u_cat[:, :D] + i_cat[:, :D]
    h = jnp.maximum(h + b0[...], 0.0)
    h = jnp.maximum(
        jnp.dot(h, w1[...], preferred_element_type=jnp.float32) + b1[...], 0.0)
    h = jnp.maximum(
        jnp.dot(h, w2[...], preferred_element_type=jnp.float32) + b2[...], 0.0)
    mf = u_cat[:, D:] * i_cat[:, D:]
    logits = jnp.dot(h, wtop[...], preferred_element_type=jnp.float32)
    logits += jnp.dot(mf, wbot[...], preferred_element_type=jnp.float32)
    logits += bout[...]
    out[...] = jax.nn.sigmoid(logits)


def _mlp_tower(u_pack, u_sel, i_pack, i_sel, b0, W1, b1, W2, b2,
               W_out, b_out):
    wtop = W_out[:16]
    wbot = W_out[16:]
    grid = B // _BS
    row_spec = pl.BlockSpec((_BS, 2 * D), lambda i: (i, 0))
    sel_spec = pl.BlockSpec((_BS, 1), lambda i: (i, 0))
    full = lambda a: pl.BlockSpec(a.shape, lambda i: (0,) * a.ndim)
    args = (u_pack, u_sel, i_pack, i_sel, b0.reshape(1, -1), W1,
            b1.reshape(1, -1), W2, b2.reshape(1, -1), wtop, wbot,
            b_out.reshape(1, 1))
    specs = [row_spec, sel_spec, row_spec, sel_spec] + [full(a) for a in args[4:]]
    return pl.pallas_call(
        _mlp_body,
        grid=(grid,),
        in_specs=specs,
        out_specs=pl.BlockSpec((_BS, 1), lambda i: (i, 0)),
        out_shape=jax.ShapeDtypeStruct((B, 1), jnp.float32),
    )(*args)


def _pack_coords(idx):
    idx = idx.astype(jnp.int32)
    blk = idx // _RW
    local = idx % _RW
    q = blk * _H + local % _H
    sel = local // _H
    return q, sel.reshape(-1, 1)


def kernel(user_indices, item_indices, emb_user_mlp, emb_item_mlp,
           emb_user_mf, emb_item_mf, W0, b0, W1, b1, W2, b2, W_out, b_out):
    user_table = _reformat(emb_user_mlp.T, emb_user_mf.T, W0[:D])
    item_table = _reformat(emb_item_mlp.T, emb_item_mf.T, W0[D:])
    uq, usel = _pack_coords(user_indices)
    iq, isel = _pack_coords(item_indices)
    u_pack, i_pack = _gather2(uq, iq, user_table, item_table)
    return _mlp_tower(u_pack, usel, i_pack, isel, b0, W1, b1, W2, b2,
                      W_out, b_out)


# speedup vs baseline: 1.0096x; 1.0096x over previous
import functools

import jax
import jax.numpy as jnp
from jax import lax
from jax.experimental import pallas as pl
from jax.experimental.pallas import tpu as pltpu
from jax.experimental.pallas import tpu_sc as plsc

B = 16384
D = 64
N = 1_000_000

_info = plsc.get_sparse_core_info()
_NC = _info.num_cores
_NS = _info.num_subcores
_NW = _NC * _NS
_BPW = B // _NW

_RW = 24576
_H = _RW // 2
_NBLK = (N + _RW - 1) // _RW
_QN = _NBLK * _H


def _reformat_body(mlp_t, mf_t, w0h, out):
    dn = (((0,), (0,)), ((), ()))

    def pack(x):
        xb = x.astype(jnp.bfloat16)
        lo = jax.lax.bitcast_convert_type(xb[:_H], jnp.int16).astype(jnp.int32)
        hi = jax.lax.bitcast_convert_type(xb[_H:], jnp.int16).astype(jnp.int32)
        return (hi << 16) | (lo & 0xFFFF)

    a = jax.lax.dot_general(mlp_t[...], w0h[...], dn,
                            preferred_element_type=jnp.float32)
    out[:, :D] = pack(a)
    r = jax.lax.broadcasted_iota(jnp.int32, (D, D), 0)
    c = jax.lax.broadcasted_iota(jnp.int32, (D, D), 1)
    eye = (r == c).astype(jnp.float32)
    b = jax.lax.dot_general(mf_t[...], eye, dn,
                            preferred_element_type=jnp.float32)
    out[:, D:] = pack(b)


def _reformat(emb_mlp_t, emb_mf_t, w0h):
    in_spec = pl.BlockSpec((D, _RW), lambda i: (0, i))
    return pl.pallas_call(
        _reformat_body,
        grid=(pl.cdiv(N, _RW),),
        in_specs=[in_spec, in_spec,
                  pl.BlockSpec((D, D), lambda i: (0, 0))],
        out_specs=pl.BlockSpec((_H, 2 * D), lambda i: (i, 0)),
        out_shape=jax.ShapeDtypeStruct((_QN, 2 * D), jnp.int32),
        compiler_params=pltpu.CompilerParams(
            vmem_limit_bytes=120 * 1024 * 1024),
    )(emb_mlp_t, emb_mf_t, w0h)


_HC = _BPW // 2


def _sc_gather2(uidx_hbm, iidx_hbm, t_user, t_item, o_user, o_item,
                uidx_v, iidx_v, rows_a, rows_b, sem_a, sem_b):
    wid = lax.axis_index("s") * _NC + lax.axis_index("c")
    base = wid * _BPW
    pltpu.sync_copy(uidx_hbm.at[pl.ds(base, _BPW)], uidx_v)
    pltpu.sync_copy(iidx_hbm.at[pl.ds(base, _BPW)], iidx_v)
    work = ((t_user, uidx_v, o_user, 0), (t_item, iidx_v, o_item, 0),
            (t_user, uidx_v, o_user, 1), (t_item, iidx_v, o_item, 1))
    bufs = (rows_a, rows_b)
    sems = (sem_a, sem_b)
    pend = [None, None]
    for slot, (table, idx_v, out, half) in enumerate(work):
        s = slot % 2
        if pend[s] is not None:
            pend[s].wait()
        cp = pltpu.async_copy(
            table.at[idx_v.at[pl.ds(half * _HC, _HC)]], bufs[s], sems[s])
        cp.wait()
        pend[s] = pltpu.async_copy(
            bufs[s], out.at[pl.ds(base + half * _HC, _HC)], sems[s])
    for p in pend:
        p.wait()


_gather2 = functools.partial(
    pl.kernel,
    mesh=plsc.VectorSubcoreMesh(core_axis_name="c", subcore_axis_name="s"),
    out_type=[jax.ShapeDtypeStruct((B, 2 * D), jnp.int32)] * 2,
    scratch_types=[
        pltpu.VMEM((_BPW,), jnp.int32),
        pltpu.VMEM((_BPW,), jnp.int32),
        pltpu.VMEM((_BPW // 2, 2 * D), jnp.int32),
        pltpu.VMEM((_BPW // 2, 2 * D), jnp.int32),
        pltpu.SemaphoreType.DMA,
        pltpu.SemaphoreType.DMA,
    ],
)(_sc_gather2)


_BS = 2048


def _unpack(packed, sel):
    lo = jax.lax.bitcast_convert_type(
        jax.lax.shift_left(packed, 16), jnp.float32)
    hi = jax.lax.bitcast_convert_type(packed & jnp.int32(-65536), jnp.float32)
    return jnp.where(sel != 0, hi, lo)


def _mlp_body(u_pack, u_sel, i_pack, i_sel, b0, w1, b1, w2, b2,
              wtop, wbot, bout, out):
    u_cat = _unpack(u_pack[...], u_sel[...])
    i_cat = _unpack(i_pack[...], i_sel[...])
    h = u_cat[:, :D] + i_cat[:, :D]
    h = jnp.maximum(h + b0[...], 0.0)
    h = jnp.maximum(
        jnp.dot(h, w1[...], preferred_element_type=jnp.float32) + b1[...], 0.0)
    h = jnp.maximum(
        jnp.dot(h, w2[...], preferred_element_type=jnp.float32) + b2[...], 0.0)
    mf = u_cat[:, D:] * i_cat[:, D:]
    logits = jnp.dot(h, wtop[...], preferred_element_type=jnp.float32)
    logits += jnp.dot(mf, wbot[...], preferred_element_type=jnp.float32)
    logits += bout[...]
    out[...] = jax.nn.sigmoid(logits)


def _mlp_tower(u_pack, u_sel, i_pack, i_sel, b0, W1, b1, W2, b2,
               W_out, b_out):
    wtop = W_out[:16]
    wbot = W_out[16:]
    grid = B // _BS
    row_spec = pl.BlockSpec((_BS, 2 * D), lambda i: (i, 0))
    sel_spec = pl.BlockSpec((_BS, 1), lambda i: (i, 0))
    full = lambda a: pl.BlockSpec(a.shape, lambda i: (0,) * a.ndim)
    args = (u_pack, u_sel, i_pack, i_sel, b0.reshape(1, -1), W1,
            b1.reshape(1, -1), W2, b2.reshape(1, -1), wtop, wbot,
            b_out.reshape(1, 1))
    specs = [row_spec, sel_spec, row_spec, sel_spec] + [full(a) for a in args[4:]]
    return pl.pallas_call(
        _mlp_body,
        grid=(grid,),
        in_specs=specs,
        out_specs=pl.BlockSpec((_BS, 1), lambda i: (i, 0)),
        out_shape=jax.ShapeDtypeStruct((B, 1), jnp.float32),
    )(*args)


def _pack_coords(idx):
    idx = idx.astype(jnp.int32)
    blk = idx // _RW
    local = idx % _RW
    q = blk * _H + local % _H
    sel = local // _H
    return q, sel.reshape(-1, 1)


def kernel(user_indices, item_indices, emb_user_mlp, emb_item_mlp,
           emb_user_mf, emb_item_mf, W0, b0, W1, b1, W2, b2, W_out, b_out):
    user_table = _reformat(emb_user_mlp.T, emb_user_mf.T, W0[:D])
    item_table = _reformat(emb_item_mlp.T, emb_item_mf.T, W0[D:])
    uq, usel = _pack_coords(user_indices)
    iq, isel = _pack_coords(item_indices)
    u_pack, i_pack = _gather2(uq, iq, user_table, item_table)
    return _mlp_tower(u_pack, usel, i_pack, isel, b0, W1, b1, W2, b2,
                      W_out, b_out)
